# baseline (speedup 1.0000x reference)
import functools

import jax
import jax.numpy as jnp
from jax.experimental import pallas as pl
from jax.experimental.pallas import tpu as pltpu

HIDDEN_DIM = 1024
MEMORY_DIM = 1024
TOTAL_SLOTS = 1040
GAMMA = 0.92
_WINDOW = 4


def _fused_kernel(x_ref, basis_ref, efac_ref, escale_ref, ediag_ref,
                  pfac_ref, pscale_ref, pdiag_ref, breadth_ref, torque_ref,
                  wr_ref, eta_ref, alpha_ref, init_re_ref, init_im_ref,
                  y_ref,
                  wa_s, wc_s, cr_s, sr_s, zre_s, zim_s, drive_s, nre_s,
                  *, bt, nb):
    c = pl.program_id(0)

    @pl.when(c == 0)
    def _init():
        basis = basis_ref[...]
        efac = efac_ref[...]
        pfac = pfac_ref[...]
        breadth = 1.0 + jnp.tanh(breadth_ref[...])
        eta = jax.nn.softplus(eta_ref[0, 0])

        b1 = basis[:, :MEMORY_DIM]
        br1 = breadth[:, :MEMORY_DIM]
        ed1 = ediag_ref[...][:, :MEMORY_DIM]

        f = jnp.dot(basis, breadth.T * efac,
                    preferred_element_type=jnp.float32)
        low = jnp.dot(f * escale_ref[...],
                      efac[:MEMORY_DIM, :].T,
                      preferred_element_type=jnp.float32)
        wa_s[...] = eta * (b1 * ((1.0 + ed1) * br1) + low)

        bp = jnp.dot(basis, pfac, preferred_element_type=jnp.float32)
        pf1 = pfac[:MEMORY_DIM, :]
        wc_s[...] = (jnp.dot(pf1 * pscale_ref[...], bp.T,
                             preferred_element_type=jnp.float32)
                     + pdiag_ref[...][:, :MEMORY_DIM].T * b1.T)

        leak = jax.nn.sigmoid(wr_ref[...][:, :MEMORY_DIM])
        tq = torque_ref[...][:, :MEMORY_DIM]
        g = GAMMA * leak
        cr_s[...] = (g * jnp.cos(tq)).reshape(8, 128)
        sr_s[...] = (g * jnp.sin(tq)).reshape(8, 128)

        zre_s[...] = jnp.broadcast_to(init_re_ref[...], (nb, 8, 128))
        zim_s[...] = jnp.broadcast_to(init_im_ref[...], (nb, 8, 128))

    x2 = x_ref[...].reshape(nb * bt, HIDDEN_DIM)
    dr = jnp.dot(x2, wa_s[...], preferred_element_type=jnp.float32)
    drive_s[...] = dr.reshape(nb, bt, 8, 128)

    cr = cr_s[...]
    sr = sr_s[...]
    kk = _WINDOW
    CN = [None, cr]
    SN = [None, sr]
    for _m in range(2, kk):
        CN.append(CN[-1] * cr - SN[-1] * sr)
        SN.append(SN[-1] * cr + CN[-2] * sr)

    def wgt(a, b):
        a, b = max(a, b), min(a, b)
        if a == 0:
            return None
        if b == 0:
            return CN[a] if a < kk else CN[kk - 1] * cr - SN[kk - 1] * sr
        return CN[a] * CN[b] + SN[a] * SN[b]

    g2 = cr * cr + sr * sr
    G2J = [None]
    for _j in range(1, kk + 1):
        G2J.append(g2 if _j == 1 else G2J[-1] * g2)

    def rsum(v):
        return jnp.sum(v, axis=(1, 2), keepdims=True)

    def _mul(w, v):
        return v if w is None else w * v

    def body(i, carry):
        zre, zim = carry
        t0 = kk * i
        ds = [None] + [drive_s[:, t0 + l] for l in range(kk)]

        prods = {}
        for l in range(1, kk + 1):
            for lp in range(l, kk + 1):
                prods[(l, lp)] = ds[l] * ds[lp]
        G = {}
        for j in range(1, kk + 1):
            for l in range(1, j + 1):
                for lp in range(l, j + 1):
                    G[(j, l, lp)] = rsum(_mul(wgt(j - l, j - lp),
                                              prods[(l, lp)]))

        zres = [zre]
        zims = [zim]
        for _j in range(kk):
            zres.append(cr * zres[-1] - sr * zims[-1])
            zims.append(sr * zres[-2] + cr * zims[-1])
        e = zre * zre + zim * zim
        av = [None] + [rsum(G2J[j] * e) for j in range(1, kk + 1)]
        zc = {}
        for j in range(1, kk + 1):
            for a in range(0, j):
                if (j, a) not in zc:
                    zc[(j, a)] = (zres[j] if a == 0
                                  else zres[j] * CN[a] + zims[j] * SN[a])
        bv = {}
        for j in range(1, kk + 1):
            for l in range(1, j + 1):
                bv[(j, l)] = rsum(zc[(j, j - l)] * ds[l])

        def norm_inv(s):
            return jnp.minimum(jax.lax.rsqrt(jnp.maximum(s, 0.0)), 1e8)

        sgs = [1.0]
        ss = [1.0]
        invs = [None]
        for j in range(1, kk + 1):
            s = av[j]
            for l in range(1, j + 1):
                bt_ = bv[(j, l)]
                s = s + 2.0 * (bt_ if l == 1 else sgs[l - 1] * bt_)
            for l in range(1, j + 1):
                gd = G[(j, l, l)]
                s = s + (gd if l == 1 else ss[l - 1] * gd)
            for l in range(1, j + 1):
                for lp in range(l + 1, j + 1):
                    go = G[(j, l, lp)]
                    coef = (sgs[lp - 1] if l == 1
                            else sgs[l - 1] * sgs[lp - 1])
                    s = s + 2.0 * coef * go
            inv = norm_inv(s)
            invs.append(inv)
            sgs.append(s * inv)
            ss.append(s)

        for j in range(1, kk + 1):
            w = zres[j]
            for l in range(1, j + 1):
                term = _mul(wgt(j - l, 0), ds[l])
                w = w + (term if l == 1 else sgs[l - 1] * term)
            w = w * invs[j]
            nre_s[:, t0 + j - 1] = w
            if j == kk:
                zre_n = w
        wim = zims[kk]
        for l in range(1, kk):
            wim = wim + (_mul(sgs[l - 1] if l > 1 else None,
                              SN[kk - l] * ds[l]))
        return zre_n, wim * invs[kk]

    zre, zim = jax.lax.fori_loop(0, bt // kk, body,
                                 (zre_s[...], zim_s[...]))
    zre_s[...] = zre
    zim_s[...] = zim

    n2 = nre_s[...].reshape(nb * bt, MEMORY_DIM)
    y = (jnp.dot(n2, wc_s[...], preferred_element_type=jnp.float32)
         + alpha_ref[0] * x2)
    y_ref[...] = y.reshape(nb, bt, HIDDEN_DIM)


def _kernel_impl(x, tape_init_re, tape_init_im, eta_raw, alpha,
                 epsilon_factor, epsilon_scale, epsilon_diag,
                 pred_factor, pred_scale, pred_diag,
                 torque_rotation, w_r, breadth_gate, basis,
                 interpret=False):
    b, t, h = x.shape
    bt = 512
    nch = t // bt
    init_re = tape_init_re[:MEMORY_DIM].reshape(8, 128)
    init_im = tape_init_im[:MEMORY_DIM].reshape(8, 128)
    full = lambda cc: tuple(0 for _ in range(2))

    kern = functools.partial(_fused_kernel, bt=bt, nb=b)
    return pl.pallas_call(
        kern,
        grid=(nch,),
        in_specs=[
            pl.BlockSpec((b, bt, h), lambda cc: (0, cc, 0)),
            pl.BlockSpec((h, TOTAL_SLOTS), full),
            pl.BlockSpec((TOTAL_SLOTS, 10), full),
            pl.BlockSpec((1, 10), full),
            pl.BlockSpec((1, TOTAL_SLOTS), full),
            pl.BlockSpec((TOTAL_SLOTS, 10), full),
            pl.BlockSpec((1, 10), full),
            pl.BlockSpec((1, TOTAL_SLOTS), full),
            pl.BlockSpec((1, TOTAL_SLOTS), full),
            pl.BlockSpec((1, TOTAL_SLOTS), full),
            pl.BlockSpec((1, TOTAL_SLOTS), full),
            pl.BlockSpec((1, 1), full),
            pl.BlockSpec(memory_space=pltpu.SMEM),
            pl.BlockSpec((8, 128), full),
            pl.BlockSpec((8, 128), full),
        ],
        out_specs=pl.BlockSpec((b, bt, h), lambda cc: (0, cc, 0)),
        out_shape=jax.ShapeDtypeStruct((b, t, h), jnp.float32),
        scratch_shapes=[
            pltpu.VMEM((MEMORY_DIM, MEMORY_DIM), jnp.float32),
            pltpu.VMEM((MEMORY_DIM, MEMORY_DIM), jnp.float32),
            pltpu.VMEM((8, 128), jnp.float32),
            pltpu.VMEM((8, 128), jnp.float32),
            pltpu.VMEM((b, 8, 128), jnp.float32),
            pltpu.VMEM((b, 8, 128), jnp.float32),
            pltpu.VMEM((b, bt, 8, 128), jnp.float32),
            pltpu.VMEM((b, bt, 8, 128), jnp.float32),
        ],
        compiler_params=pltpu.CompilerParams(
            dimension_semantics=("arbitrary",)),
        interpret=interpret,
    )(x, basis, epsilon_factor, epsilon_scale.reshape(1, -1),
      epsilon_diag.reshape(1, -1), pred_factor, pred_scale.reshape(1, -1),
      pred_diag.reshape(1, -1), breadth_gate.reshape(1, -1),
      torque_rotation.reshape(1, -1), w_r.reshape(1, -1),
      eta_raw.reshape(1, 1), alpha.reshape(1), init_re, init_im)


def kernel(x, tape_init_re, tape_init_im, eta_raw, alpha,
           epsilon_factor, epsilon_scale, epsilon_diag,
           pred_factor, pred_scale, pred_diag,
           torque_rotation, w_r, breadth_gate, basis):
    return _kernel_impl(x, tape_init_re, tape_init_im, eta_raw, alpha,
                        epsilon_factor, epsilon_scale, epsilon_diag,
                        pred_factor, pred_scale, pred_diag,
                        torque_rotation, w_r, breadth_gate, basis)
